# baseline (speedup 1.0000x reference)
import jax
import jax.numpy as jnp
from jax import lax
from jax.experimental import pallas as pl
from jax.experimental.pallas import tpu as pltpu
from jax.experimental.pallas import tpu_sc as plsc

B = 4096
NS = 26
VOCAB = 100000
D = 16
L = 50
ND = 13

NC = 2
NSUB = 16
NW = NC * NSUB
BW = B // NW
CH = 128
NS_A = 16
NS_B = NS - NS_A
BCH = B // CH
SQ_ROWS = BW * L
SQ_CHUNKS = SQ_ROWS // CH
LPAD = 64


def _seq_body(stab_hbm, sqi_hbm, sqp_hbm, out_pool_hbm,
              sqi_v, sqp_v, seq_v, pool_v, t0_v, sem):
    c = lax.axis_index("c")
    s = lax.axis_index("s")
    wid = s * NC + c

    pltpu.sync_copy(sqi_hbm.at[pl.ds(wid * SQ_CHUNKS, SQ_CHUNKS)], sqi_v)
    pltpu.sync_copy(sqp_hbm.at[pl.ds(wid * (BW // 2), BW // 2)], sqp_v)
    pltpu.sync_copy(stab_hbm.at[pl.ds(0, 1)], t0_v)
    t0 = t0_v[0, :]

    descs = []
    for j in range(SQ_CHUNKS):
        descs.append(pltpu.async_copy(
            stab_hbm.at[sqi_v.at[j]],
            seq_v.at[pl.ds(j * CH, CH)], sem))
    for d_ in descs:
        d_.wait()

    def pool_one(b, _):
        acc = seq_v[b * L, :]
        for l in range(1, L):
            acc = acc + seq_v[b * L + l, :]
        nvec = jnp.zeros((D,), jnp.int32)
        for ch in range(LPAD // D):
            ids = sqp_v[b // 2, pl.ds((b % 2) * LPAD + ch * D, D)]
            nvec = nvec + jnp.where(ids > 0, 1, 0).astype(jnp.int32)
        n = nvec[0]
        for i in range(1, D):
            n = n + nvec[i]
        nb = lax.broadcast_in_dim(n.astype(jnp.float32), (D,), ())
        pooled = (acc - (50.0 - nb) * t0) / jnp.maximum(nb, 1.0)
        pool_v[b >> 3, pl.ds((b & 7) << 4, D)] = pooled
        return 0

    lax.fori_loop(0, BW, pool_one, 0)
    pltpu.sync_copy(pool_v, out_pool_hbm.at[pl.ds(wid * (BW * D // 128),
                                                  BW * D // 128)])


def _sp_body(ppw, tab_hbm, sidx_hbm, out_t_hbm,
             idx_a, idx_b, pbuf_a, pbuf_b, sem):
    c = lax.axis_index("c")
    s = lax.axis_index("s")
    wid = s * NC + c

    bufs = [(idx_a, pbuf_a), (idx_b, pbuf_b)]
    pend = {}
    for i in range(ppw):
        sl = i % 2
        ib, pb = bufs[sl]
        if sl in pend:
            descs, pprev = pend.pop(sl)
            for d_ in descs:
                d_.wait()
            pltpu.sync_copy(pb, out_t_hbm.at[pl.ds(pprev * B, B)])
        p = wid * ppw + i
        pltpu.sync_copy(sidx_hbm.at[pl.ds(p * B, B)], ib)
        pend[sl] = ([pltpu.async_copy(tab_hbm.at[ib], pb, sem)], p)
    for sl in (ppw % 2, (ppw + 1) % 2):
        if sl in pend:
            descs, pprev = pend.pop(sl)
            for d_ in descs:
                d_.wait()
            pltpu.sync_copy(bufs[sl][1],
                            out_t_hbm.at[pl.ds(pprev * B, B)])


def _seq_kernel(seq_table, sqi, sqp):
    mesh = plsc.VectorSubcoreMesh(core_axis_name="c", subcore_axis_name="s")
    f = pl.kernel(
        _seq_body,
        out_type=[jax.ShapeDtypeStruct((B * D // 128, 128), jnp.float32)],
        mesh=mesh,
        compiler_params=pltpu.CompilerParams(use_tc_tiling_on_sc=False),
        scratch_types=[
            pltpu.VMEM((SQ_CHUNKS, CH), jnp.int32),
            pltpu.VMEM((BW // 2, 2 * LPAD), jnp.int32),
            pltpu.VMEM((SQ_ROWS, D), jnp.float32),
            pltpu.VMEM((BW * D // 128, 128), jnp.float32),
            pltpu.VMEM((1, D), jnp.float32),
            pltpu.SemaphoreType.DMA,
        ],
    )
    return f(seq_table, sqi, sqp)


def _sp_kernel(tab1d, sidx, nfields):
    import functools
    nplanes = nfields * D
    ppw = nplanes // NW
    mesh = plsc.VectorSubcoreMesh(core_axis_name="c", subcore_axis_name="s")
    f = pl.kernel(
        functools.partial(_sp_body, ppw),
        out_type=[jax.ShapeDtypeStruct((nplanes * B,), jnp.float32)],
        mesh=mesh,
        compiler_params=pltpu.CompilerParams(use_tc_tiling_on_sc=False),
        scratch_types=[
            pltpu.VMEM((B,), jnp.int32),
            pltpu.VMEM((B,), jnp.int32),
            pltpu.VMEM((B,), jnp.float32),
            pltpu.VMEM((B,), jnp.float32),
            pltpu.SemaphoreType.DMA,
        ],
    )
    return f(tab1d, sidx)


def kernel(sparse_idx, seq_idx, dense_x, sparse_tables, seq_table):
    qi = seq_idx.astype(jnp.int32)
    sqi = qi.reshape(NW * SQ_CHUNKS, CH)
    sqp = jnp.pad(qi, ((0, 0), (0, LPAD - L))).reshape(B // 2, 2 * LPAD)
    (out_pool,) = _seq_kernel(seq_table, sqi, sqp)

    sidx_t = sparse_idx.astype(jnp.int32).T

    def half(f0, nf):
        tab = jnp.transpose(sparse_tables[f0:f0 + nf], (0, 2, 1)).reshape(-1)
        offs = (jnp.arange(nf, dtype=jnp.int32)[:, None] * D
                + jnp.arange(D, dtype=jnp.int32)[None, :]) * VOCAB
        sidx = (sidx_t[f0:f0 + nf, None, :] + offs[:, :, None]).reshape(-1)
        (out,) = _sp_kernel(tab, sidx, nf)
        return out.reshape(nf, D, B)

    out_a = half(0, NS_A)
    out_b = half(NS_A, NS_B)

    sp = jnp.concatenate([out_a, out_b], axis=0).transpose(
        2, 0, 1).reshape(B, NS * D)
    return jnp.concatenate(
        [sp, out_pool.reshape(B, D), dense_x.astype(jnp.float32)], axis=1)


# speedup vs baseline: 1.2061x; 1.2061x over previous
import jax
import jax.numpy as jnp
from jax import lax
from jax.experimental import pallas as pl
from jax.experimental.pallas import tpu as pltpu
from jax.experimental.pallas import tpu_sc as plsc

B = 4096
NS = 26
VOCAB = 100000
D = 16
L = 50
ND = 13

NC = 2
NSUB = 16
NW = NC * NSUB
BW = B // NW
CH = 128
NPLANES = NS * D
PPW = NPLANES // NW
BCH = B // CH
SQ_ROWS = BW * L
SQ_CHUNKS = SQ_ROWS // CH
LPAD = 64


def _seq_body(stab_hbm, sqi_hbm, sqp_hbm, out_pool_hbm,
              sqi_v, sqp_v, seq_v, pool_v, t0_v, sem):
    c = lax.axis_index("c")
    s = lax.axis_index("s")
    wid = s * NC + c

    pltpu.sync_copy(sqi_hbm.at[pl.ds(wid * SQ_CHUNKS, SQ_CHUNKS)], sqi_v)
    pltpu.sync_copy(sqp_hbm.at[pl.ds(wid * (BW // 2), BW // 2)], sqp_v)
    pltpu.sync_copy(stab_hbm.at[pl.ds(0, 1)], t0_v)
    t0 = t0_v[0, :]

    descs = []
    for j in range(SQ_CHUNKS):
        descs.append(pltpu.async_copy(
            stab_hbm.at[sqi_v.at[j]],
            seq_v.at[pl.ds(j * CH, CH)], sem))
    for d_ in descs:
        d_.wait()

    def pool_one(b, _):
        acc = seq_v[b * L, :]
        for l in range(1, L):
            acc = acc + seq_v[b * L + l, :]
        nvec = jnp.zeros((D,), jnp.int32)
        for ch in range(LPAD // D):
            ids = sqp_v[b // 2, pl.ds((b % 2) * LPAD + ch * D, D)]
            nvec = nvec + jnp.where(ids > 0, 1, 0).astype(jnp.int32)
        n = nvec[0]
        for i in range(1, D):
            n = n + nvec[i]
        nb = lax.broadcast_in_dim(n.astype(jnp.float32), (D,), ())
        pooled = (acc - (50.0 - nb) * t0) / jnp.maximum(nb, 1.0)
        pool_v[b >> 3, pl.ds((b & 7) << 4, D)] = pooled
        return 0

    lax.fori_loop(0, BW, pool_one, 0)
    pltpu.sync_copy(pool_v, out_pool_hbm.at[pl.ds(wid * (BW * D // 128),
                                                  BW * D // 128)])


def _sp_body(ppw, tab_hbm, sidx_hbm, out_t_hbm,
             idx_a, idx_b, pbuf_a, pbuf_b, sem):
    c = lax.axis_index("c")
    s = lax.axis_index("s")
    wid = s * NC + c

    bufs = [(idx_a, pbuf_a), (idx_b, pbuf_b)]
    pend = {}
    for i in range(ppw):
        sl = i % 2
        ib, pb = bufs[sl]
        if sl in pend:
            descs, pprev = pend.pop(sl)
            for d_ in descs:
                d_.wait()
            pltpu.sync_copy(pb, out_t_hbm.at[pl.ds(pprev * B, B)])
        p = wid * ppw + i
        pltpu.sync_copy(sidx_hbm.at[pl.ds(p * B, B)], ib)
        pend[sl] = ([pltpu.async_copy(tab_hbm.at[ib], pb, sem)], p)
    for sl in (ppw % 2, (ppw + 1) % 2):
        if sl in pend:
            descs, pprev = pend.pop(sl)
            for d_ in descs:
                d_.wait()
            pltpu.sync_copy(bufs[sl][1],
                            out_t_hbm.at[pl.ds(pprev * B, B)])


def _seq_kernel(seq_table, sqi, sqp):
    mesh = plsc.VectorSubcoreMesh(core_axis_name="c", subcore_axis_name="s")
    f = pl.kernel(
        _seq_body,
        out_type=[jax.ShapeDtypeStruct((B * D // 128, 128), jnp.float32)],
        mesh=mesh,
        compiler_params=pltpu.CompilerParams(use_tc_tiling_on_sc=False),
        scratch_types=[
            pltpu.VMEM((SQ_CHUNKS, CH), jnp.int32),
            pltpu.VMEM((BW // 2, 2 * LPAD), jnp.int32),
            pltpu.VMEM((SQ_ROWS, D), jnp.float32),
            pltpu.VMEM((BW * D // 128, 128), jnp.float32),
            pltpu.VMEM((1, D), jnp.float32),
            pltpu.SemaphoreType.DMA,
        ],
    )
    return f(seq_table, sqi, sqp)


def _sp_kernel(tab1d, sidx):
    import functools
    mesh = plsc.VectorSubcoreMesh(core_axis_name="c", subcore_axis_name="s")
    f = pl.kernel(
        functools.partial(_sp_body, PPW),
        out_type=[jax.ShapeDtypeStruct((NPLANES * B,), jnp.float32)],
        mesh=mesh,
        compiler_params=pltpu.CompilerParams(use_tc_tiling_on_sc=False),
        scratch_types=[
            pltpu.VMEM((B,), jnp.int32),
            pltpu.VMEM((B,), jnp.int32),
            pltpu.VMEM((B,), jnp.float32),
            pltpu.VMEM((B,), jnp.float32),
            pltpu.SemaphoreType.DMA,
        ],
    )
    return f(tab1d, sidx)


def kernel(sparse_idx, seq_idx, dense_x, sparse_tables, seq_table):
    qi = seq_idx.astype(jnp.int32)
    sqi = qi.reshape(NW * SQ_CHUNKS, CH)
    sqp = jnp.pad(qi, ((0, 0), (0, LPAD - L))).reshape(B // 2, 2 * LPAD)
    (out_pool,) = _seq_kernel(seq_table, sqi, sqp)

    tab1d = jnp.transpose(sparse_tables, (0, 2, 1)).reshape(-1)
    sidx_t = sparse_idx.astype(jnp.int32).T
    offs = (jnp.arange(NS, dtype=jnp.int32)[:, None] * D
            + jnp.arange(D, dtype=jnp.int32)[None, :]) * VOCAB
    sidx_all = (sidx_t[:, None, :] + offs[:, :, None]).reshape(-1)
    (out_t,) = _sp_kernel(tab1d, sidx_all)

    sp = out_t.reshape(NS, D, B).transpose(2, 0, 1).reshape(B, NS * D)
    return jnp.concatenate(
        [sp, out_pool.reshape(B, D), dense_x.astype(jnp.float32)], axis=1)
